# Initial kernel scaffold; baseline (speedup 1.0000x reference)
import jax
import jax.numpy as jnp
from jax.experimental import pallas as pl


def kernel(x, edge_index, etypes, W_e, b_e, W_ih, W_hh, b_ih, b_hh):
    raise NotImplementedError("write your pallas kernel here")


# speedup vs baseline: 11.2021x; 11.2021x over previous
import functools

import jax
import jax.numpy as jnp
from jax import lax
from jax.experimental import pallas as pl
from jax.experimental.pallas import tpu as pltpu
from jax.experimental.pallas import tpu_sc as plsc

N = 10000
E = 320000
D = 128
T = 4
STEPS = 4

NC = 2
NS = 16
NW = NC * NS
EPW = E // NW
K = 125
NBLK = EPW // K
NP = 10240
RPT = NP // NS
ZR = 64

_sc_mesh = plsc.VectorSubcoreMesh(core_axis_name="c", subcore_axis_name="s",
                                  num_cores=NC, num_subcores=NS)


@functools.partial(
    pl.kernel,
    mesh=_sc_mesh,
    out_type=jax.ShapeDtypeStruct((NC, NP, D), jnp.float32),
    scratch_types=[
        pltpu.VMEM((NBLK, K), jnp.int32),
        pltpu.VMEM((NBLK, K), jnp.int32),
        pltpu.VMEM((K, D), jnp.float32),
        pltpu.VMEM((ZR, D), jnp.float32),
        pltpu.VMEM_SHARED((NP, D), jnp.float32),
        pltpu.SemaphoreType.DMA,
    ],
)
def _sc_aggregate(z_hbm, gidx_hbm, didx_hbm, out_hbm,
                  gidx_v, didx_v, rows_v, zbuf_v, acc, sem):
    cid = lax.axis_index("c")
    sid = lax.axis_index("s")
    wid = cid * NS + sid

    pltpu.sync_copy(gidx_hbm.at[wid], gidx_v)
    pltpu.sync_copy(didx_hbm.at[wid], didx_v)

    zv = jnp.zeros((16,), jnp.float32)

    def _zrow(r, carry):
        for c in range(D // 16):
            zbuf_v[r, pl.ds(c * 16, 16)] = zv
        return carry

    lax.fori_loop(0, ZR, _zrow, 0)
    for b in range(RPT // ZR):
        pltpu.sync_copy(zbuf_v, acc.at[pl.ds(sid * RPT + b * ZR, ZR)])

    plsc.subcore_barrier()

    def _eblk(j, carry):
        pltpu.async_copy(z_hbm.at[gidx_v.at[j]], rows_v, sem).wait()
        pltpu.sync_copy(rows_v, acc.at[didx_v.at[j]], add=True)
        return carry

    lax.fori_loop(0, NBLK, _eblk, 0)

    plsc.subcore_barrier()

    pltpu.sync_copy(acc.at[pl.ds(sid * RPT, RPT)],
                    out_hbm.at[cid, pl.ds(sid * RPT, RPT)])


BLK = 1000


def _transform_body(h_ref, w_ref, b_ref, z_ref):
    z_ref[...] = (jnp.dot(h_ref[...], w_ref[...],
                          preferred_element_type=jnp.float32) + b_ref[...])


def _tc_transform(h, w_cat, b_cat):
    return pl.pallas_call(
        _transform_body,
        grid=(N // BLK,),
        in_specs=[
            pl.BlockSpec((BLK, D), lambda i: (i, 0)),
            pl.BlockSpec((D, T * D), lambda i: (0, 0)),
            pl.BlockSpec((1, T * D), lambda i: (0, 0)),
        ],
        out_specs=pl.BlockSpec((BLK, T * D), lambda i: (i, 0)),
        out_shape=jax.ShapeDtypeStruct((N, T * D), jnp.float32),
    )(h, w_cat, b_cat)


def _gru_math(parts_ref, h_ref, wih_ref, whh_ref, bih_ref, bhh_ref):
    a = parts_ref[0] + parts_ref[1]
    h = h_ref[...]
    gi = jnp.dot(a, wih_ref[...], preferred_element_type=jnp.float32) + bih_ref[...]
    gh = jnp.dot(h, whh_ref[...], preferred_element_type=jnp.float32) + bhh_ref[...]
    r = jax.nn.sigmoid(gi[:, :D] + gh[:, :D])
    z = jax.nn.sigmoid(gi[:, D:2 * D] + gh[:, D:2 * D])
    cand = jnp.tanh(gi[:, 2 * D:] + r * gh[:, 2 * D:])
    return (1.0 - z) * cand + z * h


def _gru_z_body(parts_ref, h_ref, wih_ref, whh_ref, bih_ref, bhh_ref,
                wcat_ref, bcat_ref, ho_ref, zo_ref):
    hn = _gru_math(parts_ref, h_ref, wih_ref, whh_ref, bih_ref, bhh_ref)
    ho_ref[...] = hn
    zo_ref[...] = (jnp.dot(hn, wcat_ref[...],
                           preferred_element_type=jnp.float32) + bcat_ref[...])


def _gru_last_body(parts_ref, h_ref, wih_ref, whh_ref, bih_ref, bhh_ref,
                   ho_ref):
    ho_ref[...] = _gru_math(parts_ref, h_ref, wih_ref, whh_ref, bih_ref,
                            bhh_ref)


_GRU_COMMON_SPECS = [
    pl.BlockSpec((NC, BLK, D), lambda i: (0, i, 0)),
    pl.BlockSpec((BLK, D), lambda i: (i, 0)),
    pl.BlockSpec((D, 3 * D), lambda i: (0, 0)),
    pl.BlockSpec((D, 3 * D), lambda i: (0, 0)),
    pl.BlockSpec((1, 3 * D), lambda i: (0, 0)),
    pl.BlockSpec((1, 3 * D), lambda i: (0, 0)),
]


def _tc_gru_z(parts, h, wih_t, whh_t, bih, bhh, w_cat, b_cat):
    return pl.pallas_call(
        _gru_z_body,
        grid=(N // BLK,),
        in_specs=_GRU_COMMON_SPECS + [
            pl.BlockSpec((D, T * D), lambda i: (0, 0)),
            pl.BlockSpec((1, T * D), lambda i: (0, 0)),
        ],
        out_specs=[
            pl.BlockSpec((BLK, D), lambda i: (i, 0)),
            pl.BlockSpec((BLK, T * D), lambda i: (i, 0)),
        ],
        out_shape=[
            jax.ShapeDtypeStruct((N, D), jnp.float32),
            jax.ShapeDtypeStruct((N, T * D), jnp.float32),
        ],
    )(parts, h, wih_t, whh_t, bih, bhh, w_cat, b_cat)


def _tc_gru_last(parts, h, wih_t, whh_t, bih, bhh):
    return pl.pallas_call(
        _gru_last_body,
        grid=(N // BLK,),
        in_specs=_GRU_COMMON_SPECS,
        out_specs=pl.BlockSpec((BLK, D), lambda i: (i, 0)),
        out_shape=jax.ShapeDtypeStruct((N, D), jnp.float32),
    )(parts, h, wih_t, whh_t, bih, bhh)


def kernel(x, edge_index, etypes, W_e, b_e, W_ih, W_hh, b_ih, b_hh):
    w_cat = jnp.transpose(W_e, (2, 0, 1)).reshape(D, T * D)
    b_cat = b_e.reshape(1, T * D)
    wih_t = W_ih.T
    whh_t = W_hh.T
    bih = b_ih.reshape(1, 3 * D)
    bhh = b_hh.reshape(1, 3 * D)

    gidx = (edge_index[0] * T + etypes).astype(jnp.int32).reshape(NW, NBLK, K)
    didx = edge_index[1].astype(jnp.int32).reshape(NW, NBLK, K)

    h = x
    z = _tc_transform(x, w_cat, b_cat)
    for step in range(STEPS):
        parts = _sc_aggregate(z.reshape(N * T, D), gidx, didx)[:, :N]
        if step < STEPS - 1:
            h, z = _tc_gru_z(parts, h, wih_t, whh_t, bih, bhh, w_cat, b_cat)
        else:
            h = _tc_gru_last(parts, h, wih_t, whh_t, bih, bhh)
    return h
